# baseline (speedup 1.0000x reference)
import functools

import jax
import jax.numpy as jnp
from jax import lax
from jax.experimental import pallas as pl
from jax.experimental.pallas import tpu as pltpu
from jax.experimental.pallas import tpu_sc as plsc

NSEL = 300
PAD = 304
CAP = 608
MERGE = 2 * CAP
B = 16
Q = 5000
C = 80
ROW = Q * C
HALF = ROW // 2
CHUNK = 20000
NCHUNK = HALF // CHUNK
VPC = CHUNK // 16


def _km_from_f32(x):
  b = plsc.bitcast(x, jnp.int32)
  bu = plsc.bitcast(x, jnp.uint32)
  key = jnp.where(b < 0, ~bu, bu | jnp.uint32(0x80000000))
  return ~key


def _f32_from_km(km):
  key = ~km
  pos = key >= jnp.uint32(0x80000000)
  bits = jnp.where(pos, key & jnp.uint32(0x7FFFFFFF), ~key)
  return plsc.bitcast(bits, jnp.float32)


def _zero(ref, nvregs):
  z = jnp.zeros((16,), jnp.int32)

  def step(j, _):
    ref[pl.ds(j * 16, 16)] = z
    return 0

  lax.fori_loop(0, nvregs, step, 0)


def _find_bin(hist_ref, nvregs, quota):
  lanes = lax.iota(jnp.int32, 16)

  def step(j, carry):
    tot, bstar, nbelow = carry
    h = hist_ref[pl.ds(j * 16, 16)]
    cums = plsc.cumsum(h)
    hit = (tot + cums) >= quota
    hiti = hit.astype(jnp.int32)
    anyhit = jnp.sum(hiti) > 0
    lead = jnp.sum((plsc.cumsum(hiti) == 0).astype(jnp.int32))
    below_in = jnp.sum(jnp.where(lanes < lead, h, 0))
    take = jnp.logical_and(anyhit, bstar < 0)
    bstar = jnp.where(take, j * 16 + lead, bstar)
    nbelow = jnp.where(take, tot + below_in, nbelow)
    return tot + jnp.sum(h), bstar, nbelow

  _, bstar, nbelow = lax.fori_loop(0, nvregs, step, (0, -1, 0))
  return bstar, nbelow


def _radix_pass(src_km, src_idx, dst_km, dst_idx, hist_ref, shift, nbits):
  nb = 1 << nbits
  nhv = nb // 16
  dmask = jnp.uint32(nb - 1)
  _zero(hist_ref, nhv)

  def h_step(v, _):
    km = plsc.bitcast(src_km[pl.ds(v * 16, 16)], jnp.uint32)
    d = ((km >> shift) & dmask).astype(jnp.int32)
    cnt, last = plsc.scan_count(d)
    plsc.addupdate_scatter(hist_ref, [d], cnt, mask=last)
    return 0

  lax.fori_loop(0, MERGE // 16, h_step, 0)

  def s_step(j, tot):
    hv = hist_ref[pl.ds(j * 16, 16)]
    hist_ref[pl.ds(j * 16, 16)] = tot + plsc.cumsum(hv) - hv
    return tot + jnp.sum(hv)

  lax.fori_loop(0, nhv, s_step, 0)

  def p_step(v, _):
    km_i = src_km[pl.ds(v * 16, 16)]
    idx = src_idx[pl.ds(v * 16, 16)]
    km = plsc.bitcast(km_i, jnp.uint32)
    d = ((km >> shift) & dmask).astype(jnp.int32)
    off = plsc.load_gather(hist_ref, [d])
    cnt, last = plsc.scan_count(d)
    rank = off + cnt - 1
    plsc.store_scatter(dst_km, [rank], km_i)
    plsc.store_scatter(dst_idx, [rank], idx)
    plsc.addupdate_scatter(hist_ref, [d], cnt, mask=last)
    return 0

  lax.fori_loop(0, MERGE // 16, p_step, 0)


def _make_sc_kernel():
  mesh = plsc.VectorSubcoreMesh(
      core_axis_name="c", subcore_axis_name="s", num_cores=2, num_subcores=16)

  @functools.partial(
      pl.kernel,
      out_type=(
          jax.ShapeDtypeStruct((B, PAD), jnp.float32),
          jax.ShapeDtypeStruct((B, PAD), jnp.int32),
          jax.ShapeDtypeStruct((B, PAD, 4), jnp.float32),
      ),
      mesh=mesh,
      compiler_params=pltpu.CompilerParams(needs_layout_passes=False),
      scratch_types=[
          pltpu.VMEM((CHUNK,), jnp.float32),
          pltpu.VMEM((4096,), jnp.int32),
          pltpu.VMEM((CAP,), jnp.int32),
          pltpu.VMEM((CAP,), jnp.int32),
          pltpu.VMEM_SHARED((16 * CAP,), jnp.int32),
          pltpu.VMEM_SHARED((16 * CAP,), jnp.int32),
          pltpu.VMEM((MERGE,), jnp.int32),
          pltpu.VMEM((MERGE,), jnp.int32),
          pltpu.VMEM((MERGE,), jnp.int32),
          pltpu.VMEM((MERGE,), jnp.int32),
          pltpu.VMEM((32,), jnp.int32),
          pltpu.VMEM((16,), jnp.int32),
          pltpu.VMEM((PAD,), jnp.float32),
          pltpu.VMEM((PAD,), jnp.int32),
          pltpu.VMEM((PAD, 4), jnp.float32),
      ],
  )
  def body(logits_hbm, pb_hbm, ts_hbm, flip_hbm,
           out_s, out_l, out_b,
           buf, hist, cand_km, cand_idx, shared_km, shared_idx,
           m_km, m_idx, t_km, t_idx, ts_v, flip_v,
           scorebuf, labbuf, boxout):
    c = lax.axis_index("c")
    s = lax.axis_index("s")
    r = c * 8 + s // 2
    h = s % 2
    base = r * ROW + h * HALF
    lanes = lax.iota(jnp.int32, 16)

    def hist_pass(digit_fn):
      def chunk_step(ci, _):
        pltpu.sync_copy(logits_hbm.at[pl.ds(base + ci * CHUNK, CHUNK)], buf)

        @plsc.parallel_loop(0, VPC, unroll=8)
        def _v_step(v):
          km = _km_from_f32(buf[pl.ds(v * 16, 16)])
          d, valid = digit_fn(km)
          cnt, last = plsc.scan_count(d, mask=valid)
          plsc.addupdate_scatter(
              hist, [d], cnt, mask=jnp.logical_and(last, valid))

        return 0

      lax.fori_loop(0, NCHUNK, chunk_step, 0)

    all_true = jnp.full((16,), True, jnp.bool_)

    _zero(hist, 256)
    hist_pass(lambda km: ((km >> 20).astype(jnp.int32), all_true))
    b1, n1 = _find_bin(hist, 256, NSEL)
    b1u = b1.astype(jnp.uint32)
    n_bin1 = jnp.max(plsc.load_gather(hist, [jnp.full((16,), b1, jnp.int32)]))

    def fill_step(j, _):
      cand_km[pl.ds(j * 16, 16)] = jnp.full((16,), -1, jnp.int32)
      cand_idx[pl.ds(j * 16, 16)] = jnp.zeros((16,), jnp.int32)
      return 0

    lax.fori_loop(0, CAP // 16, fill_step, 0, unroll=4)

    def compact(take_fn, carry):
      def chunk_step(ci, carry):
        pltpu.sync_copy(logits_hbm.at[pl.ds(base + ci * CHUNK, CHUNK)], buf)

        def v_step(v, carry2):
          km = _km_from_f32(buf[pl.ds(v * 16, 16)])
          take, carry3, nstored_v = take_fn(km, carry2)
          dest = nstored_v + plsc.cumsum(take.astype(jnp.int32)) - 1
          gidx = h * HALF + ci * CHUNK + v * 16 + lanes
          plsc.store_scatter(cand_km, [dest], plsc.bitcast(km, jnp.int32),
                             mask=take)
          plsc.store_scatter(cand_idx, [dest], gidx, mask=take)
          return carry3

        return plsc.parallel_loop(0, VPC, unroll=4, carry=carry)(v_step)

      lax.fori_loop(0, NCHUNK, chunk_step, carry)

    def fast_path():
      def take_fn(km, carry):
        (nstored_v,) = carry
        take = (km >> 20) <= b1u
        return take, (nstored_v + plsc.all_reduce_population_count(take),), \
            nstored_v

      compact(take_fn, (jnp.zeros((16,), jnp.int32),))

    def slow_path():
      _zero(hist, 64)
      hist_pass(lambda km: (
          ((km >> 10) & jnp.uint32(0x3FF)).astype(jnp.int32),
          (km >> 20) == b1u,
      ))
      b2, n2 = _find_bin(hist, 64, NSEL - n1)
      hi22 = (b1u << 10) | b2.astype(jnp.uint32)

      _zero(hist, 64)
      hist_pass(lambda km: (
          (km & jnp.uint32(0x3FF)).astype(jnp.int32),
          (km >> 10) == hi22,
      ))
      b3, n3 = _find_bin(hist, 64, NSEL - n1 - n2)

      tkm = (hi22 << 10) | b3.astype(jnp.uint32)
      quota_eq = NSEL - n1 - n2 - n3

      def take_fn(km, carry):
        nstored_v, eqleft_v = carry
        lt = km < tkm
        eq = km == tkm
        eqpfx = plsc.cumsum(eq.astype(jnp.int32))
        take_eq = jnp.logical_and(eq, eqpfx <= eqleft_v)
        take = jnp.logical_or(lt, take_eq)
        return take, (nstored_v + plsc.all_reduce_population_count(take),
                      eqleft_v - plsc.all_reduce_population_count(take_eq)), \
            nstored_v

      compact(take_fn, (jnp.zeros((16,), jnp.int32),
                        jnp.full((16,), quota_eq, jnp.int32)))

    lax.cond(n1 + n_bin1 <= CAP, fast_path, slow_path)

    pltpu.sync_copy(cand_km, shared_km.at[pl.ds(s * CAP, CAP)])
    pltpu.sync_copy(cand_idx, shared_idx.at[pl.ds(s * CAP, CAP)])
    plsc.subcore_barrier()

    @pl.when(h == 0)
    def _merge():
      pltpu.sync_copy(shared_km.at[pl.ds(s * CAP, 2 * CAP)], m_km)
      pltpu.sync_copy(shared_idx.at[pl.ds(s * CAP, 2 * CAP)], m_idx)

      _radix_pass(m_km, m_idx, t_km, t_idx, hist, 0, 11)
      _radix_pass(t_km, t_idx, m_km, m_idx, hist, 11, 11)
      _radix_pass(m_km, m_idx, t_km, t_idx, hist, 22, 10)

      pltpu.sync_copy(ts_hbm, ts_v)
      pltpu.sync_copy(flip_hbm, flip_v)
      ih = plsc.load_gather(ts_v, [jnp.full((16,), 2 * r, jnp.int32)]
                            ).astype(jnp.float32)
      iw = plsc.load_gather(ts_v, [jnp.full((16,), 2 * r + 1, jnp.int32)]
                            ).astype(jnp.float32)
      fl = plsc.load_gather(flip_v, [jnp.full((16,), r, jnp.int32)])
      fx = jnp.logical_or(fl == 1, fl == 3)
      fy = jnp.logical_or(fl == 2, fl == 3)

      pltpu.sync_copy(pb_hbm.at[pl.ds(r * Q * 4, Q * 4)], buf)

      for i in range(PAD // 16):
        km = plsc.bitcast(t_km[pl.ds(i * 16, 16)], jnp.uint32)
        idx = t_idx[pl.ds(i * 16, 16)]
        x = _f32_from_km(km)
        scorebuf[pl.ds(i * 16, 16)] = 1.0 / (1.0 + jnp.exp(-x))
        labbuf[pl.ds(i * 16, 16)] = idx % C
        qi = (idx // C) * 4
        comp = lambda k: plsc.load_gather(buf, [qi + k])
        cx, cy, w, hh = comp(0), comp(1), comp(2), comp(3)
        x0 = (cx - 0.5 * w) * iw
        x1 = (cx + 0.5 * w) * iw
        y0 = (cy - 0.5 * hh) * ih
        y1 = (cy + 0.5 * hh) * ih
        nx0 = jnp.where(fx, iw - x1, x0)
        nx1 = jnp.where(fx, iw - x0, x1)
        ny0 = jnp.where(fy, ih - y1, y0)
        ny1 = jnp.where(fy, ih - y0, y1)
        bid = i * 16 + lanes
        plsc.store_scatter(boxout, [bid, jnp.full((16,), 0, jnp.int32)], nx0)
        plsc.store_scatter(boxout, [bid, jnp.full((16,), 1, jnp.int32)], ny0)
        plsc.store_scatter(boxout, [bid, jnp.full((16,), 2, jnp.int32)], nx1)
        plsc.store_scatter(boxout, [bid, jnp.full((16,), 3, jnp.int32)], ny1)

      pltpu.sync_copy(scorebuf, out_s.at[r])
      pltpu.sync_copy(labbuf, out_l.at[r])
      pltpu.sync_copy(boxout, out_b.at[r])

  return body


@functools.cache
def _sc_kernel():
  return _make_sc_kernel()


@jax.jit
def kernel(pred_logits, pred_boxes, target_sizes, flip):
  b, q, cc = pred_logits.shape
  assert (b, q, cc) == (B, Q, C), (b, q, cc)
  flat = pred_logits.reshape(b * q * cc)
  pb2 = pred_boxes.reshape(b * q * 4)
  ts = target_sizes.reshape(b * 2)
  scores, labels, boxes = _sc_kernel()(flat, pb2, ts, flip)
  return scores[:, :NSEL], labels[:, :NSEL], boxes[:, :NSEL]


# speedup vs baseline: 1.2759x; 1.2759x over previous
import functools

import jax
import jax.numpy as jnp
from jax import lax
from jax.experimental import pallas as pl
from jax.experimental.pallas import tpu as pltpu
from jax.experimental.pallas import tpu_sc as plsc

NSEL = 300
PAD = 304
CAP = 1024
MERGE = 2 * CAP
B = 16
Q = 5000
C = 80
ROW = Q * C
HALF = ROW // 2
CHUNK = 20000
NCHUNK = HALF // CHUNK
VPC = CHUNK // 16


def _km_from_f32(x):
  b = plsc.bitcast(x, jnp.int32)
  bu = plsc.bitcast(x, jnp.uint32)
  key = jnp.where(b < 0, ~bu, bu | jnp.uint32(0x80000000))
  return ~key


def _f32_from_km(km):
  key = ~km
  pos = key >= jnp.uint32(0x80000000)
  bits = jnp.where(pos, key & jnp.uint32(0x7FFFFFFF), ~key)
  return plsc.bitcast(bits, jnp.float32)


def _zero(ref, nvregs):
  z = jnp.zeros((16,), jnp.int32)

  def step(j, _):
    ref[pl.ds(j * 16, 16)] = z
    return 0

  lax.fori_loop(0, nvregs, step, 0)


def _find_bin(hist_ref, nvregs, quota):
  lanes = lax.iota(jnp.int32, 16)

  def step(j, carry):
    tot, bstar, nbelow = carry
    h = hist_ref[pl.ds(j * 16, 16)]
    cums = plsc.cumsum(h)
    hit = (tot + cums) >= quota
    hiti = hit.astype(jnp.int32)
    anyhit = jnp.sum(hiti) > 0
    lead = jnp.sum((plsc.cumsum(hiti) == 0).astype(jnp.int32))
    below_in = jnp.sum(jnp.where(lanes < lead, h, 0))
    take = jnp.logical_and(anyhit, bstar < 0)
    bstar = jnp.where(take, j * 16 + lead, bstar)
    nbelow = jnp.where(take, tot + below_in, nbelow)
    return tot + jnp.sum(h), bstar, nbelow

  _, bstar, nbelow = plsc.parallel_loop(
      0, nvregs, unroll=4,
      carry=(jnp.int32(0), jnp.int32(-1), jnp.int32(0)))(step)
  return bstar, nbelow


def _radix_pass(src_km, src_idx, dst_km, dst_idx, hist_ref, shift, nbits):
  nb = 1 << nbits
  nhv = nb // 16
  dmask = jnp.uint32(nb - 1)
  _zero(hist_ref, nhv)

  def h_step(v, _):
    km = plsc.bitcast(src_km[pl.ds(v * 16, 16)], jnp.uint32)
    d = ((km >> shift) & dmask).astype(jnp.int32)
    cnt, last = plsc.scan_count(d)
    plsc.addupdate_scatter(hist_ref, [d], cnt, mask=last)
    return 0

  lax.fori_loop(0, MERGE // 16, h_step, 0)

  def s_step(j, tot):
    hv = hist_ref[pl.ds(j * 16, 16)]
    hist_ref[pl.ds(j * 16, 16)] = tot + plsc.cumsum(hv) - hv
    return tot + jnp.sum(hv)

  lax.fori_loop(0, nhv, s_step, 0)

  def p_step(v, _):
    km_i = src_km[pl.ds(v * 16, 16)]
    idx = src_idx[pl.ds(v * 16, 16)]
    km = plsc.bitcast(km_i, jnp.uint32)
    d = ((km >> shift) & dmask).astype(jnp.int32)
    off = plsc.load_gather(hist_ref, [d])
    cnt, last = plsc.scan_count(d)
    rank = off + cnt - 1
    plsc.store_scatter(dst_km, [rank], km_i)
    plsc.store_scatter(dst_idx, [rank], idx)
    plsc.addupdate_scatter(hist_ref, [d], cnt, mask=last)
    return 0

  lax.fori_loop(0, MERGE // 16, p_step, 0)


def _make_sc_kernel():
  mesh = plsc.VectorSubcoreMesh(
      core_axis_name="c", subcore_axis_name="s", num_cores=2, num_subcores=16)

  @functools.partial(
      pl.kernel,
      out_type=(
          jax.ShapeDtypeStruct((B, PAD), jnp.float32),
          jax.ShapeDtypeStruct((B, PAD), jnp.int32),
          jax.ShapeDtypeStruct((B, PAD, 4), jnp.float32),
      ),
      mesh=mesh,
      compiler_params=pltpu.CompilerParams(needs_layout_passes=False),
      scratch_types=[
          pltpu.VMEM((CHUNK,), jnp.float32),
          pltpu.VMEM((8192,), jnp.int32),
          pltpu.VMEM((CAP,), jnp.int32),
          pltpu.VMEM((CAP,), jnp.int32),
          pltpu.VMEM_SHARED((16 * CAP,), jnp.int32),
          pltpu.VMEM_SHARED((16 * CAP,), jnp.int32),
          pltpu.VMEM((MERGE,), jnp.int32),
          pltpu.VMEM((MERGE,), jnp.int32),
          pltpu.VMEM((MERGE,), jnp.int32),
          pltpu.VMEM((MERGE,), jnp.int32),
          pltpu.VMEM((32,), jnp.int32),
          pltpu.VMEM((16,), jnp.int32),
          pltpu.VMEM((PAD,), jnp.float32),
          pltpu.VMEM((PAD,), jnp.int32),
          pltpu.VMEM((PAD, 4), jnp.float32),
      ],
  )
  def body(logits_hbm, pb_hbm, ts_hbm, flip_hbm,
           out_s, out_l, out_b,
           buf, hist, cand_km, cand_idx, shared_km, shared_idx,
           m_km, m_idx, t_km, t_idx, ts_v, flip_v,
           scorebuf, labbuf, boxout):
    c = lax.axis_index("c")
    s = lax.axis_index("s")
    r = c * 8 + s // 2
    h = s % 2
    base = r * ROW + h * HALF
    lanes = lax.iota(jnp.int32, 16)

    def hist_pass(digit_fn):
      def chunk_step(ci, _):
        pltpu.sync_copy(logits_hbm.at[pl.ds(base + ci * CHUNK, CHUNK)], buf)

        @plsc.parallel_loop(0, VPC, unroll=8)
        def _v_step(v):
          km = _km_from_f32(buf[pl.ds(v * 16, 16)])
          d, valid = digit_fn(km)
          cnt, last = plsc.scan_count(d, mask=valid)
          plsc.addupdate_scatter(
              hist, [d], cnt, mask=jnp.logical_and(last, valid))

        return 0

      lax.fori_loop(0, NCHUNK, chunk_step, 0)

    all_true = jnp.full((16,), True, jnp.bool_)

    _zero(hist, 512)
    hist_pass(lambda km: ((km >> 19).astype(jnp.int32), all_true))
    b1, n1 = _find_bin(hist, 512, NSEL)
    b1u = b1.astype(jnp.uint32)
    n_bin1 = jnp.max(plsc.load_gather(hist, [jnp.full((16,), b1, jnp.int32)]))

    def fill_step(j, _):
      cand_km[pl.ds(j * 16, 16)] = jnp.full((16,), -1, jnp.int32)
      cand_idx[pl.ds(j * 16, 16)] = jnp.zeros((16,), jnp.int32)
      return 0

    lax.fori_loop(0, CAP // 16, fill_step, 0, unroll=4)

    def compact(take_fn, carry):
      def chunk_step(ci, carry):
        pltpu.sync_copy(logits_hbm.at[pl.ds(base + ci * CHUNK, CHUNK)], buf)

        def v_step(v, carry2):
          km = _km_from_f32(buf[pl.ds(v * 16, 16)])
          take, carry3, nstored_v = take_fn(km, carry2)
          dest = nstored_v + plsc.cumsum(take.astype(jnp.int32)) - 1
          gidx = h * HALF + ci * CHUNK + v * 16 + lanes
          plsc.store_scatter(cand_km, [dest], plsc.bitcast(km, jnp.int32),
                             mask=take)
          plsc.store_scatter(cand_idx, [dest], gidx, mask=take)
          return carry3

        return plsc.parallel_loop(0, VPC, unroll=4, carry=carry)(v_step)

      lax.fori_loop(0, NCHUNK, chunk_step, carry)

    def fast_path():
      def take_fn(km, carry):
        (nstored_v,) = carry
        take = (km >> 19) <= b1u
        return take, (nstored_v + plsc.all_reduce_population_count(take),), \
            nstored_v

      compact(take_fn, (jnp.zeros((16,), jnp.int32),))

    def slow_path():
      _zero(hist, 64)
      hist_pass(lambda km: (
          ((km >> 9) & jnp.uint32(0x3FF)).astype(jnp.int32),
          (km >> 19) == b1u,
      ))
      b2, n2 = _find_bin(hist, 64, NSEL - n1)
      hi23 = (b1u << 10) | b2.astype(jnp.uint32)

      _zero(hist, 32)
      hist_pass(lambda km: (
          (km & jnp.uint32(0x1FF)).astype(jnp.int32),
          (km >> 9) == hi23,
      ))
      b3, n3 = _find_bin(hist, 32, NSEL - n1 - n2)

      tkm = (hi23 << 9) | b3.astype(jnp.uint32)
      quota_eq = NSEL - n1 - n2 - n3

      def take_fn(km, carry):
        nstored_v, eqleft_v = carry
        lt = km < tkm
        eq = km == tkm
        eqpfx = plsc.cumsum(eq.astype(jnp.int32))
        take_eq = jnp.logical_and(eq, eqpfx <= eqleft_v)
        take = jnp.logical_or(lt, take_eq)
        return take, (nstored_v + plsc.all_reduce_population_count(take),
                      eqleft_v - plsc.all_reduce_population_count(take_eq)), \
            nstored_v

      compact(take_fn, (jnp.zeros((16,), jnp.int32),
                        jnp.full((16,), quota_eq, jnp.int32)))

    lax.cond(n1 + n_bin1 <= CAP, fast_path, slow_path)

    pltpu.sync_copy(cand_km, shared_km.at[pl.ds(s * CAP, CAP)])
    pltpu.sync_copy(cand_idx, shared_idx.at[pl.ds(s * CAP, CAP)])
    plsc.subcore_barrier()

    @pl.when(h == 0)
    def _merge():
      pltpu.sync_copy(shared_km.at[pl.ds(s * CAP, 2 * CAP)], m_km)
      pltpu.sync_copy(shared_idx.at[pl.ds(s * CAP, 2 * CAP)], m_idx)

      _radix_pass(m_km, m_idx, t_km, t_idx, hist, 0, 11)
      _radix_pass(t_km, t_idx, m_km, m_idx, hist, 11, 11)
      _radix_pass(m_km, m_idx, t_km, t_idx, hist, 22, 10)

      pltpu.sync_copy(ts_hbm, ts_v)
      pltpu.sync_copy(flip_hbm, flip_v)
      ih = plsc.load_gather(ts_v, [jnp.full((16,), 2 * r, jnp.int32)]
                            ).astype(jnp.float32)
      iw = plsc.load_gather(ts_v, [jnp.full((16,), 2 * r + 1, jnp.int32)]
                            ).astype(jnp.float32)
      fl = plsc.load_gather(flip_v, [jnp.full((16,), r, jnp.int32)])
      fx = jnp.logical_or(fl == 1, fl == 3)
      fy = jnp.logical_or(fl == 2, fl == 3)

      pltpu.sync_copy(pb_hbm.at[pl.ds(r * Q * 4, Q * 4)], buf)

      for i in range(PAD // 16):
        km = plsc.bitcast(t_km[pl.ds(i * 16, 16)], jnp.uint32)
        idx = t_idx[pl.ds(i * 16, 16)]
        x = _f32_from_km(km)
        scorebuf[pl.ds(i * 16, 16)] = 1.0 / (1.0 + jnp.exp(-x))
        labbuf[pl.ds(i * 16, 16)] = idx % C
        qi = (idx // C) * 4
        comp = lambda k: plsc.load_gather(buf, [qi + k])
        cx, cy, w, hh = comp(0), comp(1), comp(2), comp(3)
        x0 = (cx - 0.5 * w) * iw
        x1 = (cx + 0.5 * w) * iw
        y0 = (cy - 0.5 * hh) * ih
        y1 = (cy + 0.5 * hh) * ih
        nx0 = jnp.where(fx, iw - x1, x0)
        nx1 = jnp.where(fx, iw - x0, x1)
        ny0 = jnp.where(fy, ih - y1, y0)
        ny1 = jnp.where(fy, ih - y0, y1)
        bid = i * 16 + lanes
        plsc.store_scatter(boxout, [bid, jnp.full((16,), 0, jnp.int32)], nx0)
        plsc.store_scatter(boxout, [bid, jnp.full((16,), 1, jnp.int32)], ny0)
        plsc.store_scatter(boxout, [bid, jnp.full((16,), 2, jnp.int32)], nx1)
        plsc.store_scatter(boxout, [bid, jnp.full((16,), 3, jnp.int32)], ny1)

      pltpu.sync_copy(scorebuf, out_s.at[r])
      pltpu.sync_copy(labbuf, out_l.at[r])
      pltpu.sync_copy(boxout, out_b.at[r])

  return body


@functools.cache
def _sc_kernel():
  return _make_sc_kernel()


@jax.jit
def kernel(pred_logits, pred_boxes, target_sizes, flip):
  b, q, cc = pred_logits.shape
  assert (b, q, cc) == (B, Q, C), (b, q, cc)
  flat = pred_logits.reshape(b * q * cc)
  pb2 = pred_boxes.reshape(b * q * 4)
  ts = target_sizes.reshape(b * 2)
  scores, labels, boxes = _sc_kernel()(flat, pb2, ts, flip)
  return scores[:, :NSEL], labels[:, :NSEL], boxes[:, :NSEL]


# speedup vs baseline: 1.3923x; 1.0912x over previous
import functools

import jax
import jax.numpy as jnp
from jax import lax
from jax.experimental import pallas as pl
from jax.experimental.pallas import tpu as pltpu
from jax.experimental.pallas import tpu_sc as plsc

NSEL = 300
PAD = 304
CAP = 1024
MERGE = 2 * CAP
B = 16
Q = 5000
C = 80
ROW = Q * C
HALF = ROW // 2
CHUNK = 20000
NCHUNK = HALF // CHUNK
VPC = CHUNK // 16


def _km_from_f32(x):
  b = plsc.bitcast(x, jnp.int32)
  bu = plsc.bitcast(x, jnp.uint32)
  key = jnp.where(b < 0, ~bu, bu | jnp.uint32(0x80000000))
  return ~key


def _f32_from_km(km):
  key = ~km
  pos = key >= jnp.uint32(0x80000000)
  bits = jnp.where(pos, key & jnp.uint32(0x7FFFFFFF), ~key)
  return plsc.bitcast(bits, jnp.float32)


def _zero(ref, nvregs):
  z = jnp.zeros((16,), jnp.int32)

  def step(j, _):
    ref[pl.ds(j * 16, 16)] = z
    return 0

  lax.fori_loop(0, nvregs, step, 0)


def _find_bin(hist_ref, nvregs, quota):
  lanes = lax.iota(jnp.int32, 16)

  def step(j, carry):
    tot, bstar, nbelow = carry
    h = hist_ref[pl.ds(j * 16, 16)]
    cums = plsc.cumsum(h)
    hit = (tot + cums) >= quota
    hiti = hit.astype(jnp.int32)
    anyhit = jnp.sum(hiti) > 0
    lead = jnp.sum((plsc.cumsum(hiti) == 0).astype(jnp.int32))
    below_in = jnp.sum(jnp.where(lanes < lead, h, 0))
    take = jnp.logical_and(anyhit, bstar < 0)
    bstar = jnp.where(take, j * 16 + lead, bstar)
    nbelow = jnp.where(take, tot + below_in, nbelow)
    return tot + jnp.sum(h), bstar, nbelow

  _, bstar, nbelow = plsc.parallel_loop(
      0, nvregs, unroll=4,
      carry=(jnp.int32(0), jnp.int32(-1), jnp.int32(0)))(step)
  return bstar, nbelow


def _radix_pass(src_km, src_idx, dst_km, dst_idx, hist_ref, shift, nbits):
  nb = 1 << nbits
  nhv = nb // 16
  dmask = jnp.uint32(nb - 1)
  _zero(hist_ref, nhv)

  def h_step(v, _):
    km = plsc.bitcast(src_km[pl.ds(v * 16, 16)], jnp.uint32)
    d = ((km >> shift) & dmask).astype(jnp.int32)
    cnt, last = plsc.scan_count(d)
    plsc.addupdate_scatter(hist_ref, [d], cnt, mask=last)
    return 0

  lax.fori_loop(0, MERGE // 16, h_step, 0)

  def s_step(j, tot):
    hv = hist_ref[pl.ds(j * 16, 16)]
    hist_ref[pl.ds(j * 16, 16)] = tot + plsc.cumsum(hv) - hv
    return tot + jnp.sum(hv)

  lax.fori_loop(0, nhv, s_step, 0)

  def p_step(v, _):
    km_i = src_km[pl.ds(v * 16, 16)]
    idx = src_idx[pl.ds(v * 16, 16)]
    km = plsc.bitcast(km_i, jnp.uint32)
    d = ((km >> shift) & dmask).astype(jnp.int32)
    off = plsc.load_gather(hist_ref, [d])
    cnt, last = plsc.scan_count(d)
    rank = off + cnt - 1
    plsc.store_scatter(dst_km, [rank], km_i)
    plsc.store_scatter(dst_idx, [rank], idx)
    plsc.addupdate_scatter(hist_ref, [d], cnt, mask=last)
    return 0

  lax.fori_loop(0, MERGE // 16, p_step, 0)


def _make_sc_kernel():
  mesh = plsc.VectorSubcoreMesh(
      core_axis_name="c", subcore_axis_name="s", num_cores=2, num_subcores=16)

  @functools.partial(
      pl.kernel,
      out_type=(
          jax.ShapeDtypeStruct((B, PAD), jnp.float32),
          jax.ShapeDtypeStruct((B, PAD), jnp.int32),
          jax.ShapeDtypeStruct((B, PAD, 4), jnp.float32),
      ),
      mesh=mesh,
      compiler_params=pltpu.CompilerParams(needs_layout_passes=False),
      scratch_types=[
          pltpu.VMEM((CHUNK,), jnp.float32),
          pltpu.VMEM((CHUNK,), jnp.float32),
          pltpu.SemaphoreType.DMA,
          pltpu.SemaphoreType.DMA,
          pltpu.VMEM((8192,), jnp.int32),
          pltpu.VMEM((CAP,), jnp.int32),
          pltpu.VMEM((CAP,), jnp.int32),
          pltpu.VMEM_SHARED((16 * CAP,), jnp.int32),
          pltpu.VMEM_SHARED((16 * CAP,), jnp.int32),
          pltpu.VMEM((MERGE,), jnp.int32),
          pltpu.VMEM((MERGE,), jnp.int32),
          pltpu.VMEM((MERGE,), jnp.int32),
          pltpu.VMEM((MERGE,), jnp.int32),
          pltpu.VMEM((32,), jnp.int32),
          pltpu.VMEM((16,), jnp.int32),
          pltpu.VMEM((PAD,), jnp.float32),
          pltpu.VMEM((PAD,), jnp.int32),
          pltpu.VMEM((PAD, 4), jnp.float32),
      ],
  )
  def body(logits_hbm, pb_hbm, ts_hbm, flip_hbm,
           out_s, out_l, out_b,
           buf, buf_b, sem_a, sem_b, hist, cand_km, cand_idx,
           shared_km, shared_idx,
           m_km, m_idx, t_km, t_idx, ts_v, flip_v,
           scorebuf, labbuf, boxout):
    c = lax.axis_index("c")
    s = lax.axis_index("s")
    r = c * 8 + s // 2
    h = s % 2
    base = r * ROW + h * HALF
    lanes = lax.iota(jnp.int32, 16)

    def _start(ci, dst, sem):
      pltpu.async_copy(logits_hbm.at[pl.ds(base + ci * CHUNK, CHUNK)],
                       dst, sem)

    def _drain(dst, sem):
      pltpu.make_async_copy(logits_hbm.at[pl.ds(base, CHUNK)], dst, sem
                            ).wait()

    def stream_loop(process, carry_init):
      _start(0, buf, sem_a)

      def pair_step(p, carry):
        _start(2 * p + 1, buf_b, sem_b)
        _drain(buf, sem_a)
        carry = process(buf, 2 * p, carry)

        @pl.when(p < NCHUNK // 2 - 1)
        def _():
          _start(2 * p + 2, buf, sem_a)

        _drain(buf_b, sem_b)
        return process(buf_b, 2 * p + 1, carry)

      return lax.fori_loop(0, NCHUNK // 2, pair_step, carry_init)

    def hist_pass(digit_fn):
      def process(cbuf, ci, carry):
        @plsc.parallel_loop(0, VPC, unroll=8)
        def _v_step(v):
          km = _km_from_f32(cbuf[pl.ds(v * 16, 16)])
          d, valid = digit_fn(km)
          cnt, last = plsc.scan_count(d, mask=valid)
          plsc.addupdate_scatter(
              hist, [d], cnt, mask=jnp.logical_and(last, valid))

        return carry

      stream_loop(process, jnp.int32(0))

    all_true = jnp.full((16,), True, jnp.bool_)

    _zero(hist, 512)
    hist_pass(lambda km: ((km >> 19).astype(jnp.int32), all_true))
    b1, n1 = _find_bin(hist, 512, NSEL)
    b1u = b1.astype(jnp.uint32)
    n_bin1 = jnp.max(plsc.load_gather(hist, [jnp.full((16,), b1, jnp.int32)]))

    def fill_step(j, _):
      cand_km[pl.ds(j * 16, 16)] = jnp.full((16,), -1, jnp.int32)
      cand_idx[pl.ds(j * 16, 16)] = jnp.zeros((16,), jnp.int32)
      return 0

    lax.fori_loop(0, CAP // 16, fill_step, 0, unroll=4)

    def compact(take_fn, carry):
      def process(cbuf, ci, carry2):
        def v_step(v, carry3):
          km = _km_from_f32(cbuf[pl.ds(v * 16, 16)])
          take, carry4, nstored_v = take_fn(km, carry3)
          dest = nstored_v + plsc.cumsum(take.astype(jnp.int32)) - 1
          gidx = h * HALF + ci * CHUNK + v * 16 + lanes
          plsc.store_scatter(cand_km, [dest], plsc.bitcast(km, jnp.int32),
                             mask=take)
          plsc.store_scatter(cand_idx, [dest], gidx, mask=take)
          return carry4

        return plsc.parallel_loop(0, VPC, unroll=4, carry=carry2)(v_step)

      stream_loop(process, carry)

    def fast_path():
      def take_fn(km, carry):
        (nstored_v,) = carry
        take = (km >> 19) <= b1u
        return take, (nstored_v + plsc.all_reduce_population_count(take),), \
            nstored_v

      compact(take_fn, (jnp.zeros((16,), jnp.int32),))

    def slow_path():
      _zero(hist, 64)
      hist_pass(lambda km: (
          ((km >> 9) & jnp.uint32(0x3FF)).astype(jnp.int32),
          (km >> 19) == b1u,
      ))
      b2, n2 = _find_bin(hist, 64, NSEL - n1)
      hi23 = (b1u << 10) | b2.astype(jnp.uint32)

      _zero(hist, 32)
      hist_pass(lambda km: (
          (km & jnp.uint32(0x1FF)).astype(jnp.int32),
          (km >> 9) == hi23,
      ))
      b3, n3 = _find_bin(hist, 32, NSEL - n1 - n2)

      tkm = (hi23 << 9) | b3.astype(jnp.uint32)
      quota_eq = NSEL - n1 - n2 - n3

      def take_fn(km, carry):
        nstored_v, eqleft_v = carry
        lt = km < tkm
        eq = km == tkm
        eqpfx = plsc.cumsum(eq.astype(jnp.int32))
        take_eq = jnp.logical_and(eq, eqpfx <= eqleft_v)
        take = jnp.logical_or(lt, take_eq)
        return take, (nstored_v + plsc.all_reduce_population_count(take),
                      eqleft_v - plsc.all_reduce_population_count(take_eq)), \
            nstored_v

      compact(take_fn, (jnp.zeros((16,), jnp.int32),
                        jnp.full((16,), quota_eq, jnp.int32)))

    lax.cond(n1 + n_bin1 <= CAP, fast_path, slow_path)

    pltpu.sync_copy(cand_km, shared_km.at[pl.ds(s * CAP, CAP)])
    pltpu.sync_copy(cand_idx, shared_idx.at[pl.ds(s * CAP, CAP)])
    plsc.subcore_barrier()

    @pl.when(h == 0)
    def _merge():
      pltpu.sync_copy(shared_km.at[pl.ds(s * CAP, 2 * CAP)], m_km)
      pltpu.sync_copy(shared_idx.at[pl.ds(s * CAP, 2 * CAP)], m_idx)

      _radix_pass(m_km, m_idx, t_km, t_idx, hist, 0, 11)
      _radix_pass(t_km, t_idx, m_km, m_idx, hist, 11, 11)
      _radix_pass(m_km, m_idx, t_km, t_idx, hist, 22, 10)

      pltpu.sync_copy(ts_hbm, ts_v)
      pltpu.sync_copy(flip_hbm, flip_v)
      ih = plsc.load_gather(ts_v, [jnp.full((16,), 2 * r, jnp.int32)]
                            ).astype(jnp.float32)
      iw = plsc.load_gather(ts_v, [jnp.full((16,), 2 * r + 1, jnp.int32)]
                            ).astype(jnp.float32)
      fl = plsc.load_gather(flip_v, [jnp.full((16,), r, jnp.int32)])
      fx = jnp.logical_or(fl == 1, fl == 3)
      fy = jnp.logical_or(fl == 2, fl == 3)

      pltpu.sync_copy(pb_hbm.at[pl.ds(r * Q * 4, Q * 4)], buf)

      for i in range(PAD // 16):
        km = plsc.bitcast(t_km[pl.ds(i * 16, 16)], jnp.uint32)
        idx = t_idx[pl.ds(i * 16, 16)]
        x = _f32_from_km(km)
        scorebuf[pl.ds(i * 16, 16)] = 1.0 / (1.0 + jnp.exp(-x))
        labbuf[pl.ds(i * 16, 16)] = idx % C
        qi = (idx // C) * 4
        comp = lambda k: plsc.load_gather(buf, [qi + k])
        cx, cy, w, hh = comp(0), comp(1), comp(2), comp(3)
        x0 = (cx - 0.5 * w) * iw
        x1 = (cx + 0.5 * w) * iw
        y0 = (cy - 0.5 * hh) * ih
        y1 = (cy + 0.5 * hh) * ih
        nx0 = jnp.where(fx, iw - x1, x0)
        nx1 = jnp.where(fx, iw - x0, x1)
        ny0 = jnp.where(fy, ih - y1, y0)
        ny1 = jnp.where(fy, ih - y0, y1)
        bid = i * 16 + lanes
        plsc.store_scatter(boxout, [bid, jnp.full((16,), 0, jnp.int32)], nx0)
        plsc.store_scatter(boxout, [bid, jnp.full((16,), 1, jnp.int32)], ny0)
        plsc.store_scatter(boxout, [bid, jnp.full((16,), 2, jnp.int32)], nx1)
        plsc.store_scatter(boxout, [bid, jnp.full((16,), 3, jnp.int32)], ny1)

      pltpu.sync_copy(scorebuf, out_s.at[r])
      pltpu.sync_copy(labbuf, out_l.at[r])
      pltpu.sync_copy(boxout, out_b.at[r])

  return body


@functools.cache
def _sc_kernel():
  return _make_sc_kernel()


@jax.jit
def kernel(pred_logits, pred_boxes, target_sizes, flip):
  b, q, cc = pred_logits.shape
  assert (b, q, cc) == (B, Q, C), (b, q, cc)
  flat = pred_logits.reshape(b * q * cc)
  pb2 = pred_boxes.reshape(b * q * 4)
  ts = target_sizes.reshape(b * 2)
  scores, labels, boxes = _sc_kernel()(flat, pb2, ts, flip)
  return scores[:, :NSEL], labels[:, :NSEL], boxes[:, :NSEL]
